# baseline (speedup 1.0000x reference)
import functools

import jax
import jax.numpy as jnp
from jax import lax
from jax.experimental import pallas as pl
from jax.experimental.pallas import tpu as pltpu
from jax.experimental.pallas import tpu_sc as plsc

N_ATOMS = 3_200_000
N_STRUCT = 100_000
N_SPECIES = 16

LANES = 16
NSUB = 16
ROW = 128
ROWS = N_ATOMS // ROW
CHUNK_ROWS = 48
NSLOT = 4

ACC_PAD = 100_352
HALF = ACC_PAD // 2
CSLICE = HALF // NSUB


_SC_PARAMS = pltpu.CompilerParams(needs_layout_passes=False)


LAST_N = N_STRUCT - (ACC_PAD - CSLICE)


@functools.partial(
    pl.kernel,
    mesh=plsc.VectorSubcoreMesh(core_axis_name="c", subcore_axis_name="s"),
    out_type=jax.ShapeDtypeStruct((N_STRUCT,), jnp.float32),
    scratch_types=[
        pltpu.VMEM((N_SPECIES,), jnp.float32),
        pltpu.VMEM((LANES,), jnp.int32),
        pltpu.VMEM((NSLOT, CHUNK_ROWS, ROW), jnp.int32),
        pltpu.VMEM((NSLOT, CHUNK_ROWS, ROW), jnp.int32),
        pltpu.VMEM((NSLOT, CHUNK_ROWS, ROW), jnp.float32),
        pltpu.VMEM((CSLICE,), jnp.float32),
        pltpu.VMEM((CSLICE,), jnp.float32),
        pltpu.VMEM_SHARED((ACC_PAD,), jnp.float32),
        pltpu.SemaphoreType.DMA,
        pltpu.SemaphoreType.DMA,
        pltpu.SemaphoreType.DMA,
        pltpu.SemaphoreType.DMA,
        pltpu.SemaphoreType.DMA,
        pltpu.SemaphoreType.DMA,
        pltpu.SemaphoreType.DMA,
        pltpu.SemaphoreType.DMA,
        pltpu.SemaphoreType.DMA,
    ],
    compiler_params=_SC_PARAMS,
)
def _fused(sid_hbm, sp_hbm, w_hbm, t_hbm, sv_hbm, out_hbm, wtab, svec,
           sidb, spb, valb, buf_a, buf_b, acc,
           si0, si1, si2, si3, ss0, ss1, ss2, ss3, st):
    c = lax.axis_index("c")
    s = lax.axis_index("s")
    in_sems = [si0, si1, si2, si3]
    sc_sems = [ss0, ss1, ss2, ss3]

    off = c * HALF + s * CSLICE
    is_last = jnp.logical_and(c == 1, s == NSUB - 1)

    @pl.when(is_last)
    def _():
        pltpu.async_copy(t_hbm.at[pl.ds(off, LAST_N)],
                         buf_a.at[pl.ds(0, LAST_N)], st)
    @pl.when(jnp.logical_not(is_last))
    def _():
        pltpu.async_copy(t_hbm.at[pl.ds(off, CSLICE)], buf_a, st)

    pltpu.sync_copy(w_hbm, wtab)
    pltpu.sync_copy(sv_hbm, svec)
    a_split = svec[pl.ds(0, LANES)][0]
    r_hi = (a_split + (ROW - 1)) // ROW
    r_lo = a_split // ROW

    nc0 = (r_hi + CHUNK_ROWS - 1) // CHUNK_ROWS
    nc1 = (ROWS - r_lo + CHUNK_ROWS - 1) // CHUNK_ROWS
    nc = jnp.where(c == 0, nc0, nc1)
    ncm = jnp.maximum(nc, 1)
    start_row = jnp.where(c == 0, 0, ROWS - CHUNK_ROWS * nc1)
    vt4 = ((ncm + NSUB - 1) // NSUB + NSLOT - 1) // NSLOT

    def valid(t):
        return s + NSUB * t < nc

    def chunk_rowbase(t):
        q = jnp.minimum(s + NSUB * t, ncm - 1)
        return start_row + CHUNK_ROWS * q

    def fire_in(t, j):
        base = chunk_rowbase(t)
        pltpu.async_copy(sid_hbm.at[pl.ds(base, CHUNK_ROWS)], sidb.at[j],
                         in_sems[j])
        pltpu.async_copy(sp_hbm.at[pl.ds(base, CHUNK_ROWS)], spb.at[j],
                         in_sems[j])

    def wait_in(j):
        pltpu.make_async_copy(sid_hbm.at[pl.ds(0, CHUNK_ROWS)], sidb.at[j],
                              in_sems[j]).wait()
        pltpu.make_async_copy(sp_hbm.at[pl.ds(0, CHUNK_ROWS)], spb.at[j],
                              in_sems[j]).wait()

    def compute(j):
        def row_body(r, _):
            for v in range(ROW // LANES):
                sp16 = spb[j, r, pl.ds(v * LANES, LANES)]
                valb[j, r, pl.ds(v * LANES, LANES)] = plsc.load_gather(
                    wtab, [sp16])
            return _
        lax.fori_loop(0, CHUNK_ROWS, row_body, None)

    def fire_scat(j):
        for r in range(CHUNK_ROWS):
            pltpu.async_copy(valb.at[j, r], acc.at[sidb.at[j, r]], sc_sems[j],
                             add=True)

    def drain_scat(j):
        for r in range(CHUNK_ROWS):
            pltpu.make_async_copy(valb.at[j, r], acc.at[sidb.at[j, r]],
                                  sc_sems[j]).wait()

    def visit(t, j, do_drain):
        @pl.when(valid(t))
        def _():
            wait_in(j)
            compute(j)
            fire_scat(j)
        j3 = (j + 2) % NSLOT
        if do_drain:
            @pl.when(valid(t - 2))
            def _():
                drain_scat(j3)
        @pl.when(valid(t + 2))
        def _():
            fire_in(t + 2, j3)

    @pl.when(valid(0))
    def _():
        fire_in(0, 0)
    @pl.when(valid(1))
    def _():
        fire_in(1, 1)

    def zero_body(i, _):
        buf_b[pl.ds(i * LANES, LANES)] = jnp.zeros((LANES,), jnp.float32)
        return _
    lax.fori_loop(0, CSLICE // LANES, zero_body, None)
    pltpu.sync_copy(buf_b, acc.at[pl.ds(off, CSLICE)])
    plsc.subcore_barrier()

    for j in range(NSLOT):
        visit(j, j, do_drain=(j >= 2))

    def steady_body(k, _):
        for j in range(NSLOT):
            visit(k * NSLOT + j, j, do_drain=True)
        return _
    lax.fori_loop(1, vt4, steady_body, None)

    tlast = NSLOT * vt4
    @pl.when(valid(tlast - 2))
    def _():
        drain_scat(2)
    @pl.when(valid(tlast - 1))
    def _():
        drain_scat(3)

    plsc.subcore_barrier()
    @pl.when(is_last)
    def _():
        pltpu.make_async_copy(t_hbm.at[pl.ds(off, LAST_N)],
                              buf_a.at[pl.ds(0, LAST_N)], st).wait()
    @pl.when(jnp.logical_not(is_last))
    def _():
        pltpu.make_async_copy(t_hbm.at[pl.ds(off, CSLICE)], buf_a, st).wait()
    pltpu.sync_copy(acc.at[pl.ds(off, CSLICE)], buf_b)

    def sub_body(i, _):
        d = pl.ds(i * LANES, LANES)
        buf_a[d] = buf_a[d] - buf_b[d]
        return _
    nvec = jnp.where(is_last, LAST_N // LANES, CSLICE // LANES)
    lax.fori_loop(0, nvec, sub_body, None)
    @pl.when(is_last)
    def _():
        pltpu.sync_copy(buf_a.at[pl.ds(0, LAST_N)],
                        out_hbm.at[pl.ds(off, LAST_N)])
    @pl.when(jnp.logical_not(is_last))
    def _():
        pltpu.sync_copy(buf_a, out_hbm.at[pl.ds(off, CSLICE)])


def kernel(species, structure_ids, targets, weights):
    sid2d = structure_ids.reshape(ROWS, ROW)
    sp2d = species.reshape(ROWS, ROW)
    w1 = weights.reshape(N_SPECIES)
    a_split = jnp.sum((structure_ids < HALF).astype(jnp.int32),
                      dtype=jnp.int32)
    svec = jnp.zeros((LANES,), jnp.int32).at[0].set(a_split)

    out = _fused(sid2d, sp2d, w1, targets.reshape(-1), svec)
    return out.reshape(N_STRUCT, 1)


# speedup vs baseline: 1.0892x; 1.0892x over previous
import functools

import jax
import jax.numpy as jnp
from jax import lax
from jax.experimental import pallas as pl
from jax.experimental.pallas import tpu as pltpu
from jax.experimental.pallas import tpu_sc as plsc

N_ATOMS = 3_200_000
N_STRUCT = 100_000
N_SPECIES = 16

LANES = 16
NSUB = 16
ROW = 128
ROWS = N_ATOMS // ROW
CHUNK_ROWS = 32
NSLOT = 4

ACC_PAD = 100_352
HALF = ACC_PAD // 2
CSLICE = HALF // NSUB


_SC_PARAMS = pltpu.CompilerParams(needs_layout_passes=False)


LAST_N = N_STRUCT - (ACC_PAD - CSLICE)


@functools.partial(
    pl.kernel,
    mesh=plsc.VectorSubcoreMesh(core_axis_name="c", subcore_axis_name="s"),
    out_type=jax.ShapeDtypeStruct((N_STRUCT,), jnp.float32),
    scratch_types=[
        pltpu.VMEM((N_SPECIES,), jnp.float32),
        pltpu.VMEM((LANES,), jnp.int32),
        pltpu.VMEM((NSLOT, CHUNK_ROWS, ROW), jnp.int32),
        pltpu.VMEM((NSLOT, CHUNK_ROWS, ROW), jnp.int32),
        pltpu.VMEM((NSLOT, CHUNK_ROWS, ROW), jnp.float32),
        pltpu.VMEM((CSLICE,), jnp.float32),
        pltpu.VMEM((CSLICE,), jnp.float32),
        pltpu.VMEM_SHARED((ACC_PAD,), jnp.float32),
        pltpu.SemaphoreType.DMA,
        pltpu.SemaphoreType.DMA,
        pltpu.SemaphoreType.DMA,
        pltpu.SemaphoreType.DMA,
        pltpu.SemaphoreType.DMA,
        pltpu.SemaphoreType.DMA,
        pltpu.SemaphoreType.DMA,
        pltpu.SemaphoreType.DMA,
        pltpu.SemaphoreType.DMA,
    ],
    compiler_params=_SC_PARAMS,
)
def _fused(sid_hbm, sp_hbm, w_hbm, t_hbm, sv_hbm, out_hbm, wtab, svec,
           sidb, spb, valb, buf_a, buf_b, acc,
           si0, si1, si2, si3, ss0, ss1, ss2, ss3, st):
    c = lax.axis_index("c")
    s = lax.axis_index("s")
    in_sems = [si0, si1, si2, si3]
    sc_sems = [ss0, ss1, ss2, ss3]

    off = c * HALF + s * CSLICE
    is_last = jnp.logical_and(c == 1, s == NSUB - 1)

    @pl.when(is_last)
    def _():
        pltpu.async_copy(t_hbm.at[pl.ds(off, LAST_N)],
                         buf_a.at[pl.ds(0, LAST_N)], st)
    @pl.when(jnp.logical_not(is_last))
    def _():
        pltpu.async_copy(t_hbm.at[pl.ds(off, CSLICE)], buf_a, st)

    pltpu.sync_copy(w_hbm, wtab)
    pltpu.sync_copy(sv_hbm, svec)
    a_split = svec[pl.ds(0, LANES)][0]
    r_hi = (a_split + (ROW - 1)) // ROW
    r_lo = a_split // ROW

    nc0 = (r_hi + CHUNK_ROWS - 1) // CHUNK_ROWS
    nc1 = (ROWS - r_lo + CHUNK_ROWS - 1) // CHUNK_ROWS
    nc = jnp.where(c == 0, nc0, nc1)
    ncm = jnp.maximum(nc, 1)
    start_row = jnp.where(c == 0, 0, ROWS - CHUNK_ROWS * nc1)
    vt4 = ((ncm + NSUB - 1) // NSUB + NSLOT - 1) // NSLOT

    def valid(t):
        return s + NSUB * t < nc

    def chunk_rowbase(t):
        q = jnp.minimum(s + NSUB * t, ncm - 1)
        return start_row + CHUNK_ROWS * q

    def fire_in(t, j):
        base = chunk_rowbase(t)
        pltpu.async_copy(sid_hbm.at[pl.ds(base, CHUNK_ROWS)], sidb.at[j],
                         in_sems[j])
        pltpu.async_copy(sp_hbm.at[pl.ds(base, CHUNK_ROWS)], spb.at[j],
                         in_sems[j])

    def wait_in(j):
        pltpu.make_async_copy(sid_hbm.at[pl.ds(0, CHUNK_ROWS)], sidb.at[j],
                              in_sems[j]).wait()
        pltpu.make_async_copy(sp_hbm.at[pl.ds(0, CHUNK_ROWS)], spb.at[j],
                              in_sems[j]).wait()

    def compute(j):
        def row_body(r, _):
            for v in range(ROW // LANES):
                sp16 = spb[j, r, pl.ds(v * LANES, LANES)]
                valb[j, r, pl.ds(v * LANES, LANES)] = plsc.load_gather(
                    wtab, [sp16])
            return _
        lax.fori_loop(0, CHUNK_ROWS, row_body, None)

    def fire_scat(j):
        for r in range(CHUNK_ROWS):
            pltpu.async_copy(valb.at[j, r], acc.at[sidb.at[j, r]], sc_sems[j],
                             add=True)

    def drain_scat(j):
        for r in range(CHUNK_ROWS):
            pltpu.make_async_copy(valb.at[j, r], acc.at[sidb.at[j, r]],
                                  sc_sems[j]).wait()

    def visit(t, j, do_drain):
        @pl.when(valid(t))
        def _():
            wait_in(j)
            compute(j)
            fire_scat(j)
        j3 = (j + 2) % NSLOT
        if do_drain:
            @pl.when(valid(t - 2))
            def _():
                drain_scat(j3)
        @pl.when(valid(t + 2))
        def _():
            fire_in(t + 2, j3)

    @pl.when(valid(0))
    def _():
        fire_in(0, 0)
    @pl.when(valid(1))
    def _():
        fire_in(1, 1)

    def zero_body(i, _):
        buf_b[pl.ds(i * LANES, LANES)] = jnp.zeros((LANES,), jnp.float32)
        return _
    lax.fori_loop(0, CSLICE // LANES, zero_body, None)
    pltpu.sync_copy(buf_b, acc.at[pl.ds(off, CSLICE)])
    plsc.subcore_barrier()

    for j in range(NSLOT):
        visit(j, j, do_drain=(j >= 2))

    def steady_body(k, _):
        for j in range(NSLOT):
            visit(k * NSLOT + j, j, do_drain=True)
        return _
    lax.fori_loop(1, vt4, steady_body, None)

    tlast = NSLOT * vt4
    @pl.when(valid(tlast - 2))
    def _():
        drain_scat(2)
    @pl.when(valid(tlast - 1))
    def _():
        drain_scat(3)

    plsc.subcore_barrier()
    @pl.when(is_last)
    def _():
        pltpu.make_async_copy(t_hbm.at[pl.ds(off, LAST_N)],
                              buf_a.at[pl.ds(0, LAST_N)], st).wait()
    @pl.when(jnp.logical_not(is_last))
    def _():
        pltpu.make_async_copy(t_hbm.at[pl.ds(off, CSLICE)], buf_a, st).wait()
    pltpu.sync_copy(acc.at[pl.ds(off, CSLICE)], buf_b)

    def sub_body(i, _):
        d = pl.ds(i * LANES, LANES)
        buf_a[d] = buf_a[d] - buf_b[d]
        return _
    nvec = jnp.where(is_last, LAST_N // LANES, CSLICE // LANES)
    lax.fori_loop(0, nvec, sub_body, None)
    @pl.when(is_last)
    def _():
        pltpu.sync_copy(buf_a.at[pl.ds(0, LAST_N)],
                        out_hbm.at[pl.ds(off, LAST_N)])
    @pl.when(jnp.logical_not(is_last))
    def _():
        pltpu.sync_copy(buf_a, out_hbm.at[pl.ds(off, CSLICE)])


def kernel(species, structure_ids, targets, weights):
    sid2d = structure_ids.reshape(ROWS, ROW)
    sp2d = species.reshape(ROWS, ROW)
    w1 = weights.reshape(N_SPECIES)
    a_split = jnp.sum((structure_ids < HALF).astype(jnp.int32),
                      dtype=jnp.int32)
    svec = jnp.zeros((LANES,), jnp.int32).at[0].set(a_split)

    out = _fused(sid2d, sp2d, w1, targets.reshape(-1), svec)
    return out.reshape(N_STRUCT, 1)


# speedup vs baseline: 1.1354x; 1.0425x over previous
import functools

import jax
import jax.numpy as jnp
from jax import lax
from jax.experimental import pallas as pl
from jax.experimental.pallas import tpu as pltpu
from jax.experimental.pallas import tpu_sc as plsc

N_ATOMS = 3_200_000
N_STRUCT = 100_000
N_SPECIES = 16

LANES = 16
NSUB = 16
ROW = 128
ROWS = N_ATOMS // ROW
CHUNK_ROWS = 24
NSLOT = 4

ACC_PAD = 100_352
HALF = ACC_PAD // 2
CSLICE = HALF // NSUB


_SC_PARAMS = pltpu.CompilerParams(needs_layout_passes=False)


LAST_N = N_STRUCT - (ACC_PAD - CSLICE)


@functools.partial(
    pl.kernel,
    mesh=plsc.VectorSubcoreMesh(core_axis_name="c", subcore_axis_name="s"),
    out_type=jax.ShapeDtypeStruct((N_STRUCT,), jnp.float32),
    scratch_types=[
        pltpu.VMEM((N_SPECIES,), jnp.float32),
        pltpu.VMEM((LANES,), jnp.int32),
        pltpu.VMEM((NSLOT, CHUNK_ROWS, ROW), jnp.int32),
        pltpu.VMEM((NSLOT, CHUNK_ROWS, ROW), jnp.int32),
        pltpu.VMEM((NSLOT, CHUNK_ROWS, ROW), jnp.float32),
        pltpu.VMEM((CSLICE,), jnp.float32),
        pltpu.VMEM((CSLICE,), jnp.float32),
        pltpu.VMEM_SHARED((ACC_PAD,), jnp.float32),
        pltpu.SemaphoreType.DMA,
        pltpu.SemaphoreType.DMA,
        pltpu.SemaphoreType.DMA,
        pltpu.SemaphoreType.DMA,
        pltpu.SemaphoreType.DMA,
        pltpu.SemaphoreType.DMA,
        pltpu.SemaphoreType.DMA,
        pltpu.SemaphoreType.DMA,
        pltpu.SemaphoreType.DMA,
    ],
    compiler_params=_SC_PARAMS,
)
def _fused(sid_hbm, sp_hbm, w_hbm, t_hbm, sv_hbm, out_hbm, wtab, svec,
           sidb, spb, valb, buf_a, buf_b, acc,
           si0, si1, si2, si3, ss0, ss1, ss2, ss3, st):
    c = lax.axis_index("c")
    s = lax.axis_index("s")
    in_sems = [si0, si1, si2, si3]
    sc_sems = [ss0, ss1, ss2, ss3]

    off = c * HALF + s * CSLICE
    is_last = jnp.logical_and(c == 1, s == NSUB - 1)

    @pl.when(is_last)
    def _():
        pltpu.async_copy(t_hbm.at[pl.ds(off, LAST_N)],
                         buf_a.at[pl.ds(0, LAST_N)], st)
    @pl.when(jnp.logical_not(is_last))
    def _():
        pltpu.async_copy(t_hbm.at[pl.ds(off, CSLICE)], buf_a, st)

    pltpu.sync_copy(w_hbm, wtab)
    pltpu.sync_copy(sv_hbm, svec)
    a_split = svec[pl.ds(0, LANES)][0]
    r_hi = (a_split + (ROW - 1)) // ROW
    r_lo = a_split // ROW

    nc0 = (r_hi + CHUNK_ROWS - 1) // CHUNK_ROWS
    nc1 = (ROWS - r_lo + CHUNK_ROWS - 1) // CHUNK_ROWS
    nc = jnp.where(c == 0, nc0, nc1)
    ncm = jnp.maximum(nc, 1)
    start_row = jnp.where(c == 0, 0, ROWS - CHUNK_ROWS * nc1)
    vt4 = ((ncm + NSUB - 1) // NSUB + NSLOT - 1) // NSLOT

    def valid(t):
        return s + NSUB * t < nc

    def chunk_rowbase(t):
        q = jnp.minimum(s + NSUB * t, ncm - 1)
        return start_row + CHUNK_ROWS * q

    def fire_in(t, j):
        base = chunk_rowbase(t)
        pltpu.async_copy(sid_hbm.at[pl.ds(base, CHUNK_ROWS)], sidb.at[j],
                         in_sems[j])
        pltpu.async_copy(sp_hbm.at[pl.ds(base, CHUNK_ROWS)], spb.at[j],
                         in_sems[j])

    def wait_in(j):
        pltpu.make_async_copy(sid_hbm.at[pl.ds(0, CHUNK_ROWS)], sidb.at[j],
                              in_sems[j]).wait()
        pltpu.make_async_copy(sp_hbm.at[pl.ds(0, CHUNK_ROWS)], spb.at[j],
                              in_sems[j]).wait()

    def compute(j):
        def row_body(r, _):
            for v in range(ROW // LANES):
                sp16 = spb[j, r, pl.ds(v * LANES, LANES)]
                valb[j, r, pl.ds(v * LANES, LANES)] = plsc.load_gather(
                    wtab, [sp16])
            return _
        lax.fori_loop(0, CHUNK_ROWS, row_body, None)

    def fire_scat(j):
        for r in range(CHUNK_ROWS):
            pltpu.async_copy(valb.at[j, r], acc.at[sidb.at[j, r]], sc_sems[j],
                             add=True)

    def drain_scat(j):
        for r in range(CHUNK_ROWS):
            pltpu.make_async_copy(valb.at[j, r], acc.at[sidb.at[j, r]],
                                  sc_sems[j]).wait()

    def visit(t, j, do_drain):
        @pl.when(valid(t))
        def _():
            wait_in(j)
            compute(j)
            fire_scat(j)
        j3 = (j + 2) % NSLOT
        if do_drain:
            @pl.when(valid(t - 2))
            def _():
                drain_scat(j3)
        @pl.when(valid(t + 2))
        def _():
            fire_in(t + 2, j3)

    @pl.when(valid(0))
    def _():
        fire_in(0, 0)
    @pl.when(valid(1))
    def _():
        fire_in(1, 1)

    def zero_body(i, _):
        buf_b[pl.ds(i * LANES, LANES)] = jnp.zeros((LANES,), jnp.float32)
        return _
    lax.fori_loop(0, CSLICE // LANES, zero_body, None)
    pltpu.sync_copy(buf_b, acc.at[pl.ds(off, CSLICE)])
    plsc.subcore_barrier()

    for j in range(NSLOT):
        visit(j, j, do_drain=(j >= 2))

    def steady_body(k, _):
        for j in range(NSLOT):
            visit(k * NSLOT + j, j, do_drain=True)
        return _
    lax.fori_loop(1, vt4, steady_body, None)

    tlast = NSLOT * vt4
    @pl.when(valid(tlast - 2))
    def _():
        drain_scat(2)
    @pl.when(valid(tlast - 1))
    def _():
        drain_scat(3)

    plsc.subcore_barrier()
    @pl.when(is_last)
    def _():
        pltpu.make_async_copy(t_hbm.at[pl.ds(off, LAST_N)],
                              buf_a.at[pl.ds(0, LAST_N)], st).wait()
    @pl.when(jnp.logical_not(is_last))
    def _():
        pltpu.make_async_copy(t_hbm.at[pl.ds(off, CSLICE)], buf_a, st).wait()
    pltpu.sync_copy(acc.at[pl.ds(off, CSLICE)], buf_b)

    def sub_body(i, _):
        d = pl.ds(i * LANES, LANES)
        buf_a[d] = buf_a[d] - buf_b[d]
        return _
    nvec = jnp.where(is_last, LAST_N // LANES, CSLICE // LANES)
    lax.fori_loop(0, nvec, sub_body, None)
    @pl.when(is_last)
    def _():
        pltpu.sync_copy(buf_a.at[pl.ds(0, LAST_N)],
                        out_hbm.at[pl.ds(off, LAST_N)])
    @pl.when(jnp.logical_not(is_last))
    def _():
        pltpu.sync_copy(buf_a, out_hbm.at[pl.ds(off, CSLICE)])


def kernel(species, structure_ids, targets, weights):
    sid2d = structure_ids.reshape(ROWS, ROW)
    sp2d = species.reshape(ROWS, ROW)
    w1 = weights.reshape(N_SPECIES)
    a_split = jnp.sum((structure_ids < HALF).astype(jnp.int32),
                      dtype=jnp.int32)
    svec = jnp.zeros((LANES,), jnp.int32).at[0].set(a_split)

    out = _fused(sid2d, sp2d, w1, targets.reshape(-1), svec)
    return out.reshape(N_STRUCT, 1)


# speedup vs baseline: 1.1371x; 1.0015x over previous
import functools

import jax
import jax.numpy as jnp
from jax import lax
from jax.experimental import pallas as pl
from jax.experimental.pallas import tpu as pltpu
from jax.experimental.pallas import tpu_sc as plsc

N_ATOMS = 3_200_000
N_STRUCT = 100_000
N_SPECIES = 16

LANES = 16
NSUB = 16
ROW = 128
ROWS = N_ATOMS // ROW
CHUNK_ROWS = 16
NSLOT = 4

ACC_PAD = 100_352
HALF = ACC_PAD // 2
CSLICE = HALF // NSUB


_SC_PARAMS = pltpu.CompilerParams(needs_layout_passes=False)


LAST_N = N_STRUCT - (ACC_PAD - CSLICE)


@functools.partial(
    pl.kernel,
    mesh=plsc.VectorSubcoreMesh(core_axis_name="c", subcore_axis_name="s"),
    out_type=jax.ShapeDtypeStruct((N_STRUCT,), jnp.float32),
    scratch_types=[
        pltpu.VMEM((N_SPECIES,), jnp.float32),
        pltpu.VMEM((LANES,), jnp.int32),
        pltpu.VMEM((NSLOT, CHUNK_ROWS, ROW), jnp.int32),
        pltpu.VMEM((NSLOT, CHUNK_ROWS, ROW), jnp.int32),
        pltpu.VMEM((NSLOT, CHUNK_ROWS, ROW), jnp.float32),
        pltpu.VMEM((CSLICE,), jnp.float32),
        pltpu.VMEM((CSLICE,), jnp.float32),
        pltpu.VMEM_SHARED((ACC_PAD,), jnp.float32),
        pltpu.SemaphoreType.DMA,
        pltpu.SemaphoreType.DMA,
        pltpu.SemaphoreType.DMA,
        pltpu.SemaphoreType.DMA,
        pltpu.SemaphoreType.DMA,
        pltpu.SemaphoreType.DMA,
        pltpu.SemaphoreType.DMA,
        pltpu.SemaphoreType.DMA,
        pltpu.SemaphoreType.DMA,
    ],
    compiler_params=_SC_PARAMS,
)
def _fused(sid_hbm, sp_hbm, w_hbm, t_hbm, sv_hbm, out_hbm, wtab, svec,
           sidb, spb, valb, buf_a, buf_b, acc,
           si0, si1, si2, si3, ss0, ss1, ss2, ss3, st):
    c = lax.axis_index("c")
    s = lax.axis_index("s")
    in_sems = [si0, si1, si2, si3]
    sc_sems = [ss0, ss1, ss2, ss3]

    off = c * HALF + s * CSLICE
    is_last = jnp.logical_and(c == 1, s == NSUB - 1)

    @pl.when(is_last)
    def _():
        pltpu.async_copy(t_hbm.at[pl.ds(off, LAST_N)],
                         buf_a.at[pl.ds(0, LAST_N)], st)
    @pl.when(jnp.logical_not(is_last))
    def _():
        pltpu.async_copy(t_hbm.at[pl.ds(off, CSLICE)], buf_a, st)

    pltpu.sync_copy(w_hbm, wtab)
    pltpu.sync_copy(sv_hbm, svec)
    a_split = svec[pl.ds(0, LANES)][0]
    r_hi = (a_split + (ROW - 1)) // ROW
    r_lo = a_split // ROW

    nc0 = (r_hi + CHUNK_ROWS - 1) // CHUNK_ROWS
    nc1 = (ROWS - r_lo + CHUNK_ROWS - 1) // CHUNK_ROWS
    nc = jnp.where(c == 0, nc0, nc1)
    ncm = jnp.maximum(nc, 1)
    start_row = jnp.where(c == 0, 0, ROWS - CHUNK_ROWS * nc1)
    vt4 = ((ncm + NSUB - 1) // NSUB + NSLOT - 1) // NSLOT

    def valid(t):
        return s + NSUB * t < nc

    def chunk_rowbase(t):
        q = jnp.minimum(s + NSUB * t, ncm - 1)
        return start_row + CHUNK_ROWS * q

    def fire_in(t, j):
        base = chunk_rowbase(t)
        pltpu.async_copy(sid_hbm.at[pl.ds(base, CHUNK_ROWS)], sidb.at[j],
                         in_sems[j])
        pltpu.async_copy(sp_hbm.at[pl.ds(base, CHUNK_ROWS)], spb.at[j],
                         in_sems[j])

    def wait_in(j):
        pltpu.make_async_copy(sid_hbm.at[pl.ds(0, CHUNK_ROWS)], sidb.at[j],
                              in_sems[j]).wait()
        pltpu.make_async_copy(sp_hbm.at[pl.ds(0, CHUNK_ROWS)], spb.at[j],
                              in_sems[j]).wait()

    def compute(j):
        def row_body(r, _):
            for v in range(ROW // LANES):
                sp16 = spb[j, r, pl.ds(v * LANES, LANES)]
                valb[j, r, pl.ds(v * LANES, LANES)] = plsc.load_gather(
                    wtab, [sp16])
            return _
        lax.fori_loop(0, CHUNK_ROWS, row_body, None)

    def fire_scat(j):
        for r in range(CHUNK_ROWS):
            pltpu.async_copy(valb.at[j, r], acc.at[sidb.at[j, r]], sc_sems[j],
                             add=True)

    def drain_scat(j):
        for r in range(CHUNK_ROWS):
            pltpu.make_async_copy(valb.at[j, r], acc.at[sidb.at[j, r]],
                                  sc_sems[j]).wait()

    def visit(t, j, do_drain):
        @pl.when(valid(t))
        def _():
            wait_in(j)
            compute(j)
            fire_scat(j)
        j3 = (j + 2) % NSLOT
        if do_drain:
            @pl.when(valid(t - 2))
            def _():
                drain_scat(j3)
        @pl.when(valid(t + 2))
        def _():
            fire_in(t + 2, j3)

    @pl.when(valid(0))
    def _():
        fire_in(0, 0)
    @pl.when(valid(1))
    def _():
        fire_in(1, 1)

    def zero_body(i, _):
        buf_b[pl.ds(i * LANES, LANES)] = jnp.zeros((LANES,), jnp.float32)
        return _
    lax.fori_loop(0, CSLICE // LANES, zero_body, None)
    pltpu.sync_copy(buf_b, acc.at[pl.ds(off, CSLICE)])
    plsc.subcore_barrier()

    for j in range(NSLOT):
        visit(j, j, do_drain=(j >= 2))

    def steady_body(k, _):
        for j in range(NSLOT):
            visit(k * NSLOT + j, j, do_drain=True)
        return _
    lax.fori_loop(1, vt4, steady_body, None)

    tlast = NSLOT * vt4
    @pl.when(valid(tlast - 2))
    def _():
        drain_scat(2)
    @pl.when(valid(tlast - 1))
    def _():
        drain_scat(3)

    plsc.subcore_barrier()
    @pl.when(is_last)
    def _():
        pltpu.make_async_copy(t_hbm.at[pl.ds(off, LAST_N)],
                              buf_a.at[pl.ds(0, LAST_N)], st).wait()
    @pl.when(jnp.logical_not(is_last))
    def _():
        pltpu.make_async_copy(t_hbm.at[pl.ds(off, CSLICE)], buf_a, st).wait()
    pltpu.sync_copy(acc.at[pl.ds(off, CSLICE)], buf_b)

    def sub_body(i, _):
        d = pl.ds(i * LANES, LANES)
        buf_a[d] = buf_a[d] - buf_b[d]
        return _
    nvec = jnp.where(is_last, LAST_N // LANES, CSLICE // LANES)
    lax.fori_loop(0, nvec, sub_body, None)
    @pl.when(is_last)
    def _():
        pltpu.sync_copy(buf_a.at[pl.ds(0, LAST_N)],
                        out_hbm.at[pl.ds(off, LAST_N)])
    @pl.when(jnp.logical_not(is_last))
    def _():
        pltpu.sync_copy(buf_a, out_hbm.at[pl.ds(off, CSLICE)])


def kernel(species, structure_ids, targets, weights):
    sid2d = structure_ids.reshape(ROWS, ROW)
    sp2d = species.reshape(ROWS, ROW)
    w1 = weights.reshape(N_SPECIES)
    a_split = jnp.sum((structure_ids < HALF).astype(jnp.int32),
                      dtype=jnp.int32)
    svec = jnp.zeros((LANES,), jnp.int32).at[0].set(a_split)

    out = _fused(sid2d, sp2d, w1, targets.reshape(-1), svec)
    return out.reshape(N_STRUCT, 1)


# speedup vs baseline: 1.1817x; 1.0392x over previous
import functools

import jax
import jax.numpy as jnp
from jax import lax
from jax.experimental import pallas as pl
from jax.experimental.pallas import tpu as pltpu
from jax.experimental.pallas import tpu_sc as plsc

N_ATOMS = 3_200_000
N_STRUCT = 100_000
N_SPECIES = 16

LANES = 16
NSUB = 16
ROW = 128
ROWS = N_ATOMS // ROW
CHUNK_ROWS = 8
NSLOT = 4

ACC_PAD = 100_352
HALF = ACC_PAD // 2
CSLICE = HALF // NSUB


_SC_PARAMS = pltpu.CompilerParams(needs_layout_passes=False)


LAST_N = N_STRUCT - (ACC_PAD - CSLICE)


@functools.partial(
    pl.kernel,
    mesh=plsc.VectorSubcoreMesh(core_axis_name="c", subcore_axis_name="s"),
    out_type=jax.ShapeDtypeStruct((N_STRUCT,), jnp.float32),
    scratch_types=[
        pltpu.VMEM((N_SPECIES,), jnp.float32),
        pltpu.VMEM((LANES,), jnp.int32),
        pltpu.VMEM((NSLOT, CHUNK_ROWS, ROW), jnp.int32),
        pltpu.VMEM((NSLOT, CHUNK_ROWS, ROW), jnp.int32),
        pltpu.VMEM((NSLOT, CHUNK_ROWS, ROW), jnp.float32),
        pltpu.VMEM((CSLICE,), jnp.float32),
        pltpu.VMEM((CSLICE,), jnp.float32),
        pltpu.VMEM_SHARED((ACC_PAD,), jnp.float32),
        pltpu.SemaphoreType.DMA,
        pltpu.SemaphoreType.DMA,
        pltpu.SemaphoreType.DMA,
        pltpu.SemaphoreType.DMA,
        pltpu.SemaphoreType.DMA,
        pltpu.SemaphoreType.DMA,
        pltpu.SemaphoreType.DMA,
        pltpu.SemaphoreType.DMA,
        pltpu.SemaphoreType.DMA,
    ],
    compiler_params=_SC_PARAMS,
)
def _fused(sid_hbm, sp_hbm, w_hbm, t_hbm, sv_hbm, out_hbm, wtab, svec,
           sidb, spb, valb, buf_a, buf_b, acc,
           si0, si1, si2, si3, ss0, ss1, ss2, ss3, st):
    c = lax.axis_index("c")
    s = lax.axis_index("s")
    in_sems = [si0, si1, si2, si3]
    sc_sems = [ss0, ss1, ss2, ss3]

    off = c * HALF + s * CSLICE
    is_last = jnp.logical_and(c == 1, s == NSUB - 1)

    @pl.when(is_last)
    def _():
        pltpu.async_copy(t_hbm.at[pl.ds(off, LAST_N)],
                         buf_a.at[pl.ds(0, LAST_N)], st)
    @pl.when(jnp.logical_not(is_last))
    def _():
        pltpu.async_copy(t_hbm.at[pl.ds(off, CSLICE)], buf_a, st)

    pltpu.sync_copy(w_hbm, wtab)
    pltpu.sync_copy(sv_hbm, svec)
    a_split = svec[pl.ds(0, LANES)][0]
    r_hi = (a_split + (ROW - 1)) // ROW
    r_lo = a_split // ROW

    nc0 = (r_hi + CHUNK_ROWS - 1) // CHUNK_ROWS
    nc1 = (ROWS - r_lo + CHUNK_ROWS - 1) // CHUNK_ROWS
    nc = jnp.where(c == 0, nc0, nc1)
    ncm = jnp.maximum(nc, 1)
    start_row = jnp.where(c == 0, 0, ROWS - CHUNK_ROWS * nc1)
    vt4 = ((ncm + NSUB - 1) // NSUB + NSLOT - 1) // NSLOT

    def valid(t):
        return s + NSUB * t < nc

    def chunk_rowbase(t):
        q = jnp.minimum(s + NSUB * t, ncm - 1)
        return start_row + CHUNK_ROWS * q

    def fire_in(t, j):
        base = chunk_rowbase(t)
        pltpu.async_copy(sid_hbm.at[pl.ds(base, CHUNK_ROWS)], sidb.at[j],
                         in_sems[j])
        pltpu.async_copy(sp_hbm.at[pl.ds(base, CHUNK_ROWS)], spb.at[j],
                         in_sems[j])

    def wait_in(j):
        pltpu.make_async_copy(sid_hbm.at[pl.ds(0, CHUNK_ROWS)], sidb.at[j],
                              in_sems[j]).wait()
        pltpu.make_async_copy(sp_hbm.at[pl.ds(0, CHUNK_ROWS)], spb.at[j],
                              in_sems[j]).wait()

    def compute(j):
        def row_body(r, _):
            for v in range(ROW // LANES):
                sp16 = spb[j, r, pl.ds(v * LANES, LANES)]
                valb[j, r, pl.ds(v * LANES, LANES)] = plsc.load_gather(
                    wtab, [sp16])
            return _
        lax.fori_loop(0, CHUNK_ROWS, row_body, None)

    def fire_scat(j):
        for r in range(CHUNK_ROWS):
            pltpu.async_copy(valb.at[j, r], acc.at[sidb.at[j, r]], sc_sems[j],
                             add=True)

    def drain_scat(j):
        for r in range(CHUNK_ROWS):
            pltpu.make_async_copy(valb.at[j, r], acc.at[sidb.at[j, r]],
                                  sc_sems[j]).wait()

    def visit(t, j, do_drain):
        @pl.when(valid(t))
        def _():
            wait_in(j)
            compute(j)
            fire_scat(j)
        j3 = (j + 2) % NSLOT
        if do_drain:
            @pl.when(valid(t - 2))
            def _():
                drain_scat(j3)
        @pl.when(valid(t + 2))
        def _():
            fire_in(t + 2, j3)

    @pl.when(valid(0))
    def _():
        fire_in(0, 0)
    @pl.when(valid(1))
    def _():
        fire_in(1, 1)

    def zero_body(i, _):
        buf_b[pl.ds(i * LANES, LANES)] = jnp.zeros((LANES,), jnp.float32)
        return _
    lax.fori_loop(0, CSLICE // LANES, zero_body, None)
    pltpu.sync_copy(buf_b, acc.at[pl.ds(off, CSLICE)])
    plsc.subcore_barrier()

    for j in range(NSLOT):
        visit(j, j, do_drain=(j >= 2))

    def steady_body(k, _):
        for j in range(NSLOT):
            visit(k * NSLOT + j, j, do_drain=True)
        return _
    lax.fori_loop(1, vt4, steady_body, None)

    tlast = NSLOT * vt4
    @pl.when(valid(tlast - 2))
    def _():
        drain_scat(2)
    @pl.when(valid(tlast - 1))
    def _():
        drain_scat(3)

    plsc.subcore_barrier()
    @pl.when(is_last)
    def _():
        pltpu.make_async_copy(t_hbm.at[pl.ds(off, LAST_N)],
                              buf_a.at[pl.ds(0, LAST_N)], st).wait()
    @pl.when(jnp.logical_not(is_last))
    def _():
        pltpu.make_async_copy(t_hbm.at[pl.ds(off, CSLICE)], buf_a, st).wait()
    pltpu.sync_copy(acc.at[pl.ds(off, CSLICE)], buf_b)

    def sub_body(i, _):
        d = pl.ds(i * LANES, LANES)
        buf_a[d] = buf_a[d] - buf_b[d]
        return _
    nvec = jnp.where(is_last, LAST_N // LANES, CSLICE // LANES)
    lax.fori_loop(0, nvec, sub_body, None)
    @pl.when(is_last)
    def _():
        pltpu.sync_copy(buf_a.at[pl.ds(0, LAST_N)],
                        out_hbm.at[pl.ds(off, LAST_N)])
    @pl.when(jnp.logical_not(is_last))
    def _():
        pltpu.sync_copy(buf_a, out_hbm.at[pl.ds(off, CSLICE)])


def kernel(species, structure_ids, targets, weights):
    sid2d = structure_ids.reshape(ROWS, ROW)
    sp2d = species.reshape(ROWS, ROW)
    w1 = weights.reshape(N_SPECIES)
    a_split = jnp.sum((structure_ids < HALF).astype(jnp.int32),
                      dtype=jnp.int32)
    svec = jnp.zeros((LANES,), jnp.int32).at[0].set(a_split)

    out = _fused(sid2d, sp2d, w1, targets.reshape(-1), svec)
    return out.reshape(N_STRUCT, 1)
